# baseline (speedup 1.0000x reference)
import functools

import jax
import jax.numpy as jnp
from jax import lax
from jax.experimental import pallas as pl
from jax.experimental.pallas import tpu as pltpu
from jax.experimental.pallas import tpu_sc as plsc

_N_NODE = 1000000
_B = 16384
_R = 4
_D = 32

_NC = 2
_NS = 16
_NW = _NC * _NS
_BPW = _B // _NW


def _sc_gather_body(ht_tab, rel_tab, idx_ht, idx_rel, ht_out, rel_out,
                    idx_ht_v, idx_rel_v, ht_v, rel_v, sem1, sem2):
    wid = lax.axis_index("s") * _NC + lax.axis_index("c")
    base = wid * _BPW
    pltpu.sync_copy(idx_ht.at[pl.ds(base, _BPW)], idx_ht_v)
    pltpu.sync_copy(idx_rel.at[pl.ds(base * _R, _BPW * _R)], idx_rel_v)
    cp1 = pltpu.async_copy(ht_tab.at[idx_ht_v], ht_v, sem1)
    cp2 = pltpu.async_copy(rel_tab.at[idx_rel_v], rel_v, sem2)
    cp1.wait()
    cp2.wait()
    pltpu.sync_copy(ht_v, ht_out.at[pl.ds(base, _BPW)])
    pltpu.sync_copy(rel_v, rel_out.at[pl.ds(base * _R, _BPW * _R)])


def _sc_gather(ht_tab, rel_tab, idx_ht, idx_rel):
    mesh = plsc.VectorSubcoreMesh(core_axis_name="c", subcore_axis_name="s")
    fn = pl.kernel(
        _sc_gather_body,
        mesh=mesh,
        out_type=(
            jax.ShapeDtypeStruct((_B, _D), jnp.float32),
            jax.ShapeDtypeStruct((_B * _R, _D), jnp.float32),
        ),
        scratch_types=[
            pltpu.VMEM((_BPW,), jnp.int32),
            pltpu.VMEM((_BPW * _R,), jnp.int32),
            pltpu.VMEM((_BPW, _D), jnp.float32),
            pltpu.VMEM((_BPW * _R, _D), jnp.float32),
            pltpu.SemaphoreType.DMA,
            pltpu.SemaphoreType.DMA,
        ],
        compiler_params=pltpu.CompilerParams(use_tc_tiling_on_sc=False),
    )
    return fn(ht_tab, rel_tab, idx_ht, idx_rel)


def _tc_loss_body(ht_ref, rel_ref, gt_ref, out_ref):
    i = pl.program_id(0)
    ht = ht_ref[...]
    s_ht = jnp.sum(ht * ht, axis=-1, keepdims=True)
    inv_ht = 1.0 / jnp.maximum(jnp.sqrt(s_ht), 1e-12)
    total = jnp.zeros((1, 1), jnp.float32)
    for r in range(_R):
        rel = rel_ref[:, r * _D:(r + 1) * _D]
        dot = jnp.sum(rel * ht, axis=-1, keepdims=True)
        ss = jnp.sum(rel * rel, axis=-1, keepdims=True)
        inv_rel = 1.0 / jnp.maximum(jnp.sqrt(ss), 1e-12)
        score = dot * inv_rel * inv_ht
        z = -score * gt_ref[:, r:r + 1]
        loss = jnp.maximum(z, 0.0) + jnp.log1p(jnp.exp(-jnp.abs(z)))
        total = total + jnp.sum(loss, axis=0, keepdims=True)

    @pl.when(i == 0)
    def _():
        out_ref[...] = jnp.zeros((1, 1), jnp.float32)

    out_ref[...] += total * (1.0 / (_B * _R))


def _tc_loss(ht_rows, rel_rows, gt):
    nb = 1024
    grid = _B // nb
    return pl.pallas_call(
        _tc_loss_body,
        grid=(grid,),
        in_specs=[
            pl.BlockSpec((nb, _D), lambda i: (i, 0)),
            pl.BlockSpec((nb, _R * _D), lambda i: (i, 0)),
            pl.BlockSpec((nb, _R), lambda i: (i, 0)),
        ],
        out_specs=pl.BlockSpec((1, 1), lambda i: (0, 0)),
        out_shape=jax.ShapeDtypeStruct((1, 1), jnp.float32),
    )(ht_rows, rel_rows, gt)


def kernel(hyper_node_embeddings, base, base_edge_index, ground_truth, rel_table):
    idx_ht = (jnp.reshape(base_edge_index, (_B,)) - _N_NODE).astype(jnp.int32)
    idx_rel = jnp.reshape(base, (_B * _R,)).astype(jnp.int32)
    ht_rows, rel_rows = _sc_gather(hyper_node_embeddings, rel_table, idx_ht, idx_rel)
    out = _tc_loss(ht_rows, jnp.reshape(rel_rows, (_B, _R * _D)), ground_truth)
    return out[0, 0]
